# Initial kernel scaffold; baseline (speedup 1.0000x reference)
import jax
import jax.numpy as jnp
from jax.experimental import pallas as pl


def kernel(values, ref, index):
    raise NotImplementedError("write your pallas kernel here")


# speedup vs baseline: 11.5849x; 11.5849x over previous
import functools

import jax
import jax.numpy as jnp
from jax import lax
from jax.experimental import pallas as pl
from jax.experimental.pallas import tpu as pltpu
from jax.experimental.pallas import tpu_sc as plsc

_M = 512

_GATHER_DNUMS = lax.GatherDimensionNumbers(
    offset_dims=(), collapsed_slice_dims=(0,), start_index_map=(0,))


def _g16(x, i):
  return lax.gather(x, i[:, None], _GATHER_DNUMS, (1,),
                    mode=lax.GatherScatterMode.PROMISE_IN_BOUNDS)


@functools.lru_cache(maxsize=None)
def _build(B, S, H):
  assert S % 16 == 0 and _M % 16 == 0
  NW = 32
  rows_per = B // NW
  n_chunks = S // 16
  m_chunks = _M // 16
  Z = 64

  mesh = plsc.VectorSubcoreMesh(core_axis_name="c", subcore_axis_name="s")

  @functools.partial(
      pl.kernel,
      mesh=mesh,
      compiler_params=pltpu.CompilerParams(
          needs_layout_passes=False, use_tc_tiling_on_sc=False),
      out_type=[
          jax.ShapeDtypeStruct((B, _M), jnp.float32),
          jax.ShapeDtypeStruct((B, _M), jnp.float32),
          jax.ShapeDtypeStruct((B * _M, H), jnp.float32),
      ],
      scratch_types=[
          pltpu.VMEM((S,), jnp.int32),
          pltpu.VMEM((S,), jnp.float32),
          pltpu.VMEM((_M,), jnp.float32),
          pltpu.VMEM((_M,), jnp.float32),
          pltpu.VMEM((_M,), jnp.float32),
          pltpu.VMEM((_M,), jnp.int32),
          pltpu.VMEM((_M,), jnp.float32),
          pltpu.VMEM((_M,), jnp.float32),
          pltpu.VMEM((_M,), jnp.int32),
          pltpu.VMEM((_M // Z, Z), jnp.int32),
          pltpu.VMEM((_M + Z, H), jnp.float32),
          pltpu.VMEM((2 * Z, H), jnp.float32),
          pltpu.SemaphoreType.DMA,
          pltpu.SemaphoreType.DMA,
      ],
  )
  def sc_kernel(values_hbm, ref_hbm, index_hbm, mean_hbm, max_hbm, gat_hbm,
                idx_v, val_v, sum_v, cnt_v, mx_v, arg_v, mean_v, mxo_v,
                src_perm, dst_perm, rows_v, zeros_nm, gsem, zsem):
    wid = lax.axis_index("s") * 2 + lax.axis_index("c")
    lane = lax.iota(jnp.int32, 16)
    zf16 = jnp.zeros((16,), jnp.float32)
    zi16 = jnp.zeros((16,), jnp.int32)
    ones16 = jnp.ones((16,), jnp.float32)
    ninf16 = jnp.full((16,), -jnp.inf, jnp.float32)

    for r in range(2 * Z):
      for cc in range(H // 16):
        zeros_nm[r, pl.ds(cc * 16, 16)] = zf16

    def row_body(k, _):
      b = wid * rows_per + k

      pltpu.sync_copy(index_hbm.at[b], idx_v)
      pltpu.sync_copy(values_hbm.at[b], val_v)

      def init_body(i, _):
        o = pl.multiple_of(i * 16, 16)
        sum_v[pl.ds(o, 16)] = zf16
        cnt_v[pl.ds(o, 16)] = zf16
        mx_v[pl.ds(o, 16)] = ninf16
        return 0

      lax.fori_loop(0, m_chunks, init_body, 0, unroll=False)

      def chunk_body(c, _):
        base = pl.multiple_of(c * 16, 16)
        idx = idx_v[pl.ds(base, 16)]
        v = val_v[pl.ds(base, 16)]
        key = idx * 16 + lane
        skey = plsc.sort_key_val(key, lane)
        skey, perm = skey
        sidx = lax.shift_right_logical(skey, 4)
        sv = _g16(v, perm)
        sp = perm
        for sh in (1, 2, 4, 8):
          pl_lane = jnp.maximum(lane - sh, 0)
          pv = _g16(sv, pl_lane)
          pp = _g16(sp, pl_lane)
          pi = _g16(sidx, pl_lane)
          valid = (lane >= sh) & (pi == sidx)
          take = valid & (pv > sv)
          sv = jnp.where(take, pv, sv)
          sp = jnp.where(take, pp, sp)
        nxt = _g16(sidx, jnp.minimum(lane + 1, 15))
        leader = (lane == 15) | (sidx != nxt)
        am = plsc.load_gather(mx_v, [sidx], mask=leader)
        aa = plsc.load_gather(arg_v, [sidx], mask=leader)
        ge = sv >= am
        nm = jnp.where(ge, sv, am)
        na = jnp.where(ge, base + sp, aa)
        plsc.store_scatter(mx_v, [sidx], nm, mask=leader)
        plsc.store_scatter(arg_v, [sidx], na, mask=leader)
        plsc.addupdate_scatter(sum_v, [idx], v)
        plsc.addupdate_scatter(cnt_v, [idx], ones16)
        return 0

      lax.fori_loop(0, n_chunks, chunk_body, 0, unroll=False)

      n_nz = jnp.int32(0)
      n_e = jnp.int32(0)
      row0 = b * S
      out0 = b * _M
      for c in range(m_chunks):
        o = c * 16
        s = sum_v[pl.ds(o, 16)]
        n = cnt_v[pl.ds(o, 16)]
        m = mx_v[pl.ds(o, 16)]
        a = arg_v[pl.ds(o, 16)]
        nz = n > 0.0
        mean_v[pl.ds(o, 16)] = s / jnp.maximum(n, 1.0)
        mxo_v[pl.ds(o, 16)] = jnp.where(nz, m, 0.0)
        ones_nz = jnp.where(nz, 1, zi16)
        ones_e = jnp.where(nz, zi16, 1)
        offs = jnp.where(nz, (n_nz - 1) + plsc.cumsum(ones_nz),
                         (_M - n_e) - plsc.cumsum(ones_e))
        plsc.store_scatter(src_perm, [offs], jnp.where(nz, row0 + a, row0))
        plsc.store_scatter(
            dst_perm,
            [lax.shift_right_logical(offs, 6), lax.bitwise_and(offs, Z - 1)],
            out0 + o + lane)
        n_nz = n_nz + jnp.sum(ones_nz)
        n_e = n_e + jnp.sum(ones_e)

      zd = [
          pltpu.async_copy(zeros_nm, gat_hbm.at[pl.ds(out0 + q * 2 * Z, 2 * Z)],
                           zsem)
          for q in range(_M // (2 * Z))
      ]
      for j in range(_M // 128):
        @pl.when(n_nz > j * 128)
        def _():
          pltpu.async_copy(
              ref_hbm.at[src_perm.at[pl.ds(j * 128, 128)]],
              rows_v.at[pl.ds(j * 128, 128)], gsem).wait()
      for d in zd:
        d.wait()
      pltpu.sync_copy(gat_hbm.at[pl.ds(out0, Z)], rows_v.at[pl.ds(n_nz, Z)])

      for r in range(_M // Z):
        @pl.when(n_nz > r * Z)
        def _():
          pltpu.async_copy(rows_v.at[pl.ds(r * Z, Z)],
                           gat_hbm.at[dst_perm.at[r]], zsem).wait()

      pltpu.sync_copy(mean_v, mean_hbm.at[b])
      pltpu.sync_copy(mxo_v, max_hbm.at[b])

      return 0

    lax.fori_loop(0, rows_per, row_body, 0, unroll=False)

  return sc_kernel


def kernel(values, ref, index):
  B, S = values.shape
  H = ref.shape[-1]
  ref_flat = ref.reshape(B * S, H)
  mean, mx, gat = _build(B, S, H)(values, ref_flat, index.astype(jnp.int32))
  return mean, mx, gat.reshape(B, _M, H)


# speedup vs baseline: 13.3139x; 1.1492x over previous
import functools

import jax
import jax.numpy as jnp
from jax import lax
from jax.experimental import pallas as pl
from jax.experimental.pallas import tpu as pltpu
from jax.experimental.pallas import tpu_sc as plsc

_M = 512

_GATHER_DNUMS = lax.GatherDimensionNumbers(
    offset_dims=(), collapsed_slice_dims=(0,), start_index_map=(0,))


def _g16(x, i):
  return lax.gather(x, i[:, None], _GATHER_DNUMS, (1,),
                    mode=lax.GatherScatterMode.PROMISE_IN_BOUNDS)


@functools.lru_cache(maxsize=None)
def _build(B, S, H):
  assert S % 16 == 0 and _M % 16 == 0
  NW = 32
  rows_per = B // NW
  n_chunks = S // 16
  m_chunks = _M // 16
  Z = 64

  mesh = plsc.VectorSubcoreMesh(core_axis_name="c", subcore_axis_name="s")

  @functools.partial(
      pl.kernel,
      mesh=mesh,
      compiler_params=pltpu.CompilerParams(
          needs_layout_passes=False, use_tc_tiling_on_sc=False),
      out_type=[
          jax.ShapeDtypeStruct((B, _M), jnp.float32),
          jax.ShapeDtypeStruct((B, _M), jnp.float32),
          jax.ShapeDtypeStruct((B * _M, H), jnp.float32),
          jax.ShapeDtypeStruct((Z, H), jnp.float32),
      ],
      scratch_types=[
          pltpu.VMEM((S,), jnp.int32),
          pltpu.VMEM((S,), jnp.float32),
          pltpu.VMEM((_M,), jnp.float32),
          pltpu.VMEM((_M,), jnp.float32),
          pltpu.VMEM((_M,), jnp.float32),
          pltpu.VMEM((_M,), jnp.int32),
          pltpu.VMEM((_M,), jnp.float32),
          pltpu.VMEM((_M,), jnp.float32),
          pltpu.VMEM((_M,), jnp.int32),
          pltpu.VMEM((_M // Z, Z), jnp.int32),
          pltpu.VMEM((_M + Z, H), jnp.float32),
          pltpu.VMEM((Z, H), jnp.float32),
          pltpu.SemaphoreType.DMA,
          pltpu.SemaphoreType.DMA,
      ],
  )
  def sc_kernel(values_hbm, ref_hbm, index_hbm, mean_hbm, max_hbm, gat_hbm,
                zeros_hbm, idx_v, val_v, sum_v, cnt_v, mx_v, arg_v, mean_v,
                mxo_v, src_perm, dst_perm, rows_v, zeros_nm, gsem, zsem):
    sid = lax.axis_index("s")
    wid = sid * 2 + lax.axis_index("c")
    lane = lax.iota(jnp.int32, 16)
    zf16 = jnp.zeros((16,), jnp.float32)
    zi16 = jnp.zeros((16,), jnp.int32)
    ones16 = jnp.ones((16,), jnp.float32)
    ninf16 = jnp.full((16,), -jnp.inf, jnp.float32)

    for r in range(Z):
      for cc in range(H // 16):
        zeros_nm[r, pl.ds(cc * 16, 16)] = zf16
    pltpu.sync_copy(zeros_nm.at[pl.ds(4 * sid, 4)],
                    zeros_hbm.at[pl.ds(4 * sid, 4)])
    plsc.subcore_barrier()

    def row_body(k, _):
      b = wid * rows_per + k

      pltpu.async_copy(index_hbm.at[b], idx_v, gsem)
      pltpu.async_copy(values_hbm.at[b], val_v, gsem)
      pltpu.make_async_copy(index_hbm.at[b], idx_v, gsem).wait()
      pltpu.make_async_copy(values_hbm.at[b], val_v, gsem).wait()

      def init_body(i, _):
        o = pl.multiple_of(i * 16, 16)
        sum_v[pl.ds(o, 16)] = zf16
        cnt_v[pl.ds(o, 16)] = zf16
        mx_v[pl.ds(o, 16)] = ninf16
        return 0

      lax.fori_loop(0, m_chunks, init_body, 0, unroll=False)

      def chunk_body(c, _):
        base = pl.multiple_of(c * 16, 16)
        idx = idx_v[pl.ds(base, 16)]
        v = val_v[pl.ds(base, 16)]
        key = idx * 16 + lane
        skey = plsc.sort_key_val(key, lane)
        skey, perm = skey
        sidx = lax.shift_right_logical(skey, 4)
        sv = _g16(v, perm)
        sp = perm
        for sh in (1, 2, 4, 8):
          pl_lane = jnp.maximum(lane - sh, 0)
          pv = _g16(sv, pl_lane)
          pp = _g16(sp, pl_lane)
          pi = _g16(sidx, pl_lane)
          valid = (lane >= sh) & (pi == sidx)
          take = valid & (pv > sv)
          sv = jnp.where(take, pv, sv)
          sp = jnp.where(take, pp, sp)
        nxt = _g16(sidx, jnp.minimum(lane + 1, 15))
        leader = (lane == 15) | (sidx != nxt)
        am = plsc.load_gather(mx_v, [sidx], mask=leader)
        aa = plsc.load_gather(arg_v, [sidx], mask=leader)
        ge = sv >= am
        nm = jnp.where(ge, sv, am)
        na = jnp.where(ge, base + sp, aa)
        plsc.store_scatter(mx_v, [sidx], nm, mask=leader)
        plsc.store_scatter(arg_v, [sidx], na, mask=leader)
        plsc.addupdate_scatter(sum_v, [idx], v)
        plsc.addupdate_scatter(cnt_v, [idx], ones16)
        return 0

      lax.fori_loop(0, n_chunks, chunk_body, 0, unroll=False)

      n_nz = jnp.int32(0)
      n_e = jnp.int32(0)
      row0 = b * S
      out0 = b * _M
      for c in range(m_chunks):
        o = c * 16
        s = sum_v[pl.ds(o, 16)]
        n = cnt_v[pl.ds(o, 16)]
        m = mx_v[pl.ds(o, 16)]
        a = arg_v[pl.ds(o, 16)]
        nz = n > 0.0
        mean_v[pl.ds(o, 16)] = s / jnp.maximum(n, 1.0)
        mxo_v[pl.ds(o, 16)] = jnp.where(nz, m, 0.0)
        ones_nz = jnp.where(nz, 1, zi16)
        ones_e = jnp.where(nz, zi16, 1)
        offs = jnp.where(nz, (n_nz - 1) + plsc.cumsum(ones_nz),
                         (_M - n_e) - plsc.cumsum(ones_e))
        plsc.store_scatter(src_perm, [offs], jnp.where(nz, row0 + a, row0))
        plsc.store_scatter(
            dst_perm,
            [lax.shift_right_logical(offs, 6), lax.bitwise_and(offs, Z - 1)],
            out0 + o + lane)
        n_nz = n_nz + jnp.sum(ones_nz)
        n_e = n_e + jnp.sum(ones_e)

      for j in range(_M // 128):
        @pl.when(n_nz > j * 128)
        def _():
          pltpu.async_copy(
              ref_hbm.at[src_perm.at[pl.ds(j * 128, 128)]],
              rows_v.at[pl.ds(j * 128, 128)], gsem)
      for j in range(_M // 128):
        @pl.when(n_nz > j * 128)
        def _():
          pltpu.make_async_copy(
              ref_hbm.at[src_perm.at[pl.ds(j * 128, 128)]],
              rows_v.at[pl.ds(j * 128, 128)], gsem).wait()
      pltpu.sync_copy(zeros_hbm, rows_v.at[pl.ds(n_nz, Z)])

      for r in range(_M // Z):
        occ = n_nz > r * Z

        @pl.when(occ)
        def _():
          pltpu.async_copy(rows_v.at[pl.ds(r * Z, Z)],
                           gat_hbm.at[dst_perm.at[r]], zsem)

        @pl.when(jnp.logical_not(occ))
        def _():
          pltpu.async_copy(zeros_nm, gat_hbm.at[dst_perm.at[r]], zsem)

      pltpu.async_copy(mean_v, mean_hbm.at[b], zsem)
      pltpu.async_copy(mxo_v, max_hbm.at[b], zsem)

      for r in range(_M // Z):
        occ = n_nz > r * Z

        @pl.when(occ)
        def _():
          pltpu.make_async_copy(rows_v.at[pl.ds(r * Z, Z)],
                                gat_hbm.at[dst_perm.at[r]], zsem).wait()

        @pl.when(jnp.logical_not(occ))
        def _():
          pltpu.make_async_copy(zeros_nm, gat_hbm.at[dst_perm.at[r]],
                                zsem).wait()

      pltpu.make_async_copy(mean_v, mean_hbm.at[b], zsem).wait()
      pltpu.make_async_copy(mxo_v, max_hbm.at[b], zsem).wait()

      return 0

    lax.fori_loop(0, rows_per, row_body, 0, unroll=False)

  return sc_kernel


def kernel(values, ref, index):
  B, S = values.shape
  H = ref.shape[-1]
  ref_flat = ref.reshape(B * S, H)
  mean, mx, gat, _ = _build(B, S, H)(values, ref_flat, index.astype(jnp.int32))
  return mean, mx, gat.reshape(B, _M, H)
